# Initial kernel scaffold; baseline (speedup 1.0000x reference)
import jax
import jax.numpy as jnp
from jax.experimental import pallas as pl


def kernel(feat, delay, is_po, edge_index, level_ids, Wpi1, bpi1, Wpi2, bpi2, Ws1, bs1, Ws2, bs2, Wn1, bn1, Wn2, bn2):
    raise NotImplementedError("write your pallas kernel here")


# speedup vs baseline: 4.3607x; 4.3607x over previous
import functools

import jax
import jax.numpy as jnp
from jax import lax
from jax.experimental import pallas as pl
from jax.experimental.pallas import tpu as pltpu
from jax.experimental.pallas import tpu_sc as plsc

N = 10000
E = 320000
HID = 128
NLVL = 8

NC = 2
NS = 16
NW = NC * NS
EPW = E // NW
CHUNK = 80
NCHUNK = EPW // CHUNK
NP = 10240
RPS = NP // NS
ZR = 128

_mesh = plsc.VectorSubcoreMesh(
    core_axis_name="c", subcore_axis_name="s", num_cores=NC, num_subcores=NS)


@functools.partial(
    pl.kernel,
    out_type=jax.ShapeDtypeStruct((NC, NP, HID), jnp.float32),
    mesh=_mesh,
    scratch_types=[
        pltpu.VMEM((CHUNK,), jnp.int32),
        pltpu.VMEM((CHUNK,), jnp.int32),
        pltpu.VMEM((CHUNK, HID), jnp.float32),
        pltpu.VMEM((ZR, HID), jnp.float32),
        pltpu.VMEM_SHARED((NP, HID), jnp.float32),
        pltpu.SemaphoreType.DMA,
    ],
)
def _sc_agg(h_hbm, src_hbm, dst_hbm, out_hbm,
            src_v, dst_v, rows_v, zero_v, acc_sh, sem):
    c = lax.axis_index("c")
    s = lax.axis_index("s")
    wid = s * NC + c

    def _zrow(i, _):
        for j in range(HID // 16):
            zero_v[i, pl.ds(j * 16, 16)] = jnp.zeros((16,), jnp.float32)
        return 0
    lax.fori_loop(0, ZR, _zrow, 0)

    def _zcpy(i, _):
        pltpu.sync_copy(zero_v, acc_sh.at[pl.ds(s * RPS + i * ZR, ZR)])
        return 0
    lax.fori_loop(0, RPS // ZR, _zcpy, 0)
    plsc.subcore_barrier()

    base = wid * EPW

    def _chunk(k, _):
        off = base + k * CHUNK
        pltpu.sync_copy(src_hbm.at[pl.ds(off, CHUNK)], src_v)
        pltpu.sync_copy(dst_hbm.at[pl.ds(off, CHUNK)], dst_v)
        pltpu.async_copy(h_hbm.at[src_v], rows_v, sem).wait()
        pltpu.sync_copy(rows_v, acc_sh.at[dst_v], add=True)
        return 0
    lax.fori_loop(0, NCHUNK, _chunk, 0)
    plsc.subcore_barrier()

    pltpu.sync_copy(acc_sh.at[pl.ds(s * RPS, RPS)],
                    out_hbm.at[c, pl.ds(s * RPS, RPS)])


_RB = 1000


def _leaky(x):
    return jnp.where(x >= 0, x, 0.1 * x)


def _tc_init_body(feat, delay, lvl, Wpi1, bpi1, Wpi2, bpi2, Ws1, bs1, Ws2,
                  bs2, h0_out, hself_out):
    hs = jnp.dot(feat[...], Ws1[...], preferred_element_type=jnp.float32)
    hs = _leaky(hs + bs1[...])
    hs = jnp.dot(hs, Ws2[...], preferred_element_type=jnp.float32) + bs2[...]
    hself_out[...] = hs

    hp = delay[...] * Wpi1[...]
    hp = _leaky(hp + bpi1[...])
    hp = jnp.dot(hp, Wpi2[...], preferred_element_type=jnp.float32) + bpi2[...]
    h0_out[...] = jnp.where(lvl[...] == 0, hp, 0.0)


def _tc_init(feat, delay, lvl2d, Wpi1, bpi1, Wpi2, bpi2, Ws1, bs1, Ws2, bs2):
    grid = (N // _RB,)
    full = lambda shape: pl.BlockSpec(shape, lambda i: (0, 0))
    row = lambda w: pl.BlockSpec((_RB, w), lambda i: (i, 0))
    return pl.pallas_call(
        _tc_init_body,
        grid=grid,
        in_specs=[row(HID), row(1), row(1),
                  full((1, 64)), full((1, 64)), full((64, HID)), full((1, HID)),
                  full((HID, 64)), full((1, 64)), full((64, HID)), full((1, HID))],
        out_specs=[row(HID), row(HID)],
        out_shape=[jax.ShapeDtypeStruct((N, HID), jnp.float32),
                   jax.ShapeDtypeStruct((N, HID), jnp.float32)],
    )(feat, delay, lvl2d, Wpi1, bpi1, Wpi2, bpi2, Ws1, bs1, Ws2, bs2)


def _tc_level_body(lref, accA, accB, degA, degB, hself, h_in, lvl, ispo,
                   Wn1, bn1, Wn2, bn2, h_out):
    lv = lref[0, 0]
    deg = jnp.maximum(degA[...] + degB[...], 1.0)
    neigh = (accA[...] + accB[...]) / deg
    hid = jnp.dot(neigh, Wn1[...], preferred_element_type=jnp.float32)
    hid = _leaky(hid + bn1[...])
    out = jnp.dot(hid, Wn2[...], preferred_element_type=jnp.float32) + bn2[...]
    out = out + hself[...]
    out = jnp.where(ispo[...] != 1, jnp.maximum(out, 0.0), out)
    h_out[...] = jnp.where(lvl[...] == lv, out, h_in[...])


def _tc_level(lval, accA, accB, degA, degB, hself, h, lvl2d, ispo,
              Wn1, bn1, Wn2, bn2):
    grid = (N // _RB,)
    full = lambda shape: pl.BlockSpec(shape, lambda i: (0, 0))
    row = lambda w: pl.BlockSpec((_RB, w), lambda i: (i, 0))
    return pl.pallas_call(
        _tc_level_body,
        grid=grid,
        in_specs=[pl.BlockSpec(memory_space=pltpu.SMEM),
                  row(HID), row(HID), row(1), row(1), row(HID), row(HID),
                  row(1), row(1),
                  full((HID, 64)), full((1, 64)), full((64, HID)), full((1, HID))],
        out_specs=row(HID),
        out_shape=jax.ShapeDtypeStruct((N, HID), jnp.float32),
    )(lval, accA, accB, degA, degB, hself, h, lvl2d, ispo, Wn1, bn1, Wn2, bn2)


def kernel(feat, delay, is_po, edge_index, level_ids, Wpi1, bpi1, Wpi2, bpi2,
           Ws1, bs1, Ws2, bs2, Wn1, bn1, Wn2, bn2):
    src = edge_index[0]
    dst = edge_index[1]
    lvl2d = level_ids[:, None]

    h, h_self = _tc_init(feat, delay, lvl2d,
                         Wpi1, bpi1[None, :], Wpi2, bpi2[None, :],
                         Ws1, bs1[None, :], Ws2, bs2[None, :])

    ones_tab = jnp.ones((N, HID), jnp.float32)
    degp = _sc_agg(ones_tab, src, dst)
    degA = degp[0, :N, 0:1]
    degB = degp[1, :N, 0:1]

    bn1r = bn1[None, :]
    bn2r = bn2[None, :]
    for l in range(1, NLVL):
        acc = _sc_agg(h, src, dst)
        lval = jnp.full((1, 1), l, dtype=jnp.int32)
        h = _tc_level(lval, acc[0, :N], acc[1, :N], degA, degB, h_self, h, lvl2d,
                      is_po, Wn1, bn1r, Wn2, bn2r)
    return h


# speedup vs baseline: 11.7588x; 2.6965x over previous
import functools

import jax
import jax.numpy as jnp
from jax import lax
from jax.experimental import pallas as pl
from jax.experimental.pallas import tpu as pltpu
from jax.experimental.pallas import tpu_sc as plsc

N = 10000
E = 320000
HID = 128
NLVL = 8

NC = 2
NS = 16
NW = NC * NS
EPW = E // NW
CHUNK = 80
NGRP = EPW // 16
NP = 10240
RPS = NP // NS
ZR = 128
TRASH = NP - 8
EP = E + CHUNK * 256
STAG = EPW + CHUNK

_mesh = plsc.VectorSubcoreMesh(
    core_axis_name="c", subcore_axis_name="s", num_cores=NC, num_subcores=NS)


def _iota16():
    return lax.iota(jnp.int32, 16)


@functools.partial(
    pl.kernel,
    out_type=jax.ShapeDtypeStruct((NW, 8, 128), jnp.int32),
    mesh=_mesh,
    compiler_params=pltpu.CompilerParams(needs_layout_passes=False),
    scratch_types=[
        pltpu.VMEM((EPW,), jnp.int32),
        pltpu.VMEM((N,), jnp.int32),
        pltpu.VMEM((8, 128), jnp.int32),
        pltpu.SemaphoreType.DMA,
    ],
)
def _sc_pass1(dst_hbm, lvl_hbm, cnt_hbm, dstblk, lvl_v, cblk, sem):
    c = lax.axis_index("c")
    s = lax.axis_index("s")
    wid = s * NC + c
    pltpu.sync_copy(dst_hbm.at[pl.ds(pl.multiple_of(wid * EPW, 8), EPW)], dstblk)
    pltpu.sync_copy(lvl_hbm, lvl_v)

    for r in range(8):
        for q in range(8):
            cblk[r, pl.ds(q * 16, 16)] = jnp.zeros((16,), jnp.int32)

    def _grp(g, cnt):
        d16 = dstblk[pl.ds(g * 16, 16)]
        dlev = plsc.load_gather(lvl_v, [d16])
        for l in range(1, 8):
            m = dlev == l
            pc = plsc.all_reduce_population_count(m)
            cnt = cnt + jnp.where(_iota16() == l, pc, 0)
        return cnt
    cnt = lax.fori_loop(0, NGRP, _grp, jnp.zeros((16,), jnp.int32))
    cblk[0, pl.ds(0, 16)] = cnt
    pltpu.sync_copy(cblk, cnt_hbm.at[wid])


def _bucket_layout(cnts_v, wid):
    total = jnp.zeros((16,), jnp.int32)
    mine = jnp.zeros((16,), jnp.int32)
    widv = jnp.zeros((16,), jnp.int32) + wid
    for wo in range(NW):
        row = cnts_v[wo, 0, pl.ds(0, 16)]
        pcw = ((row + (CHUNK - 1)) // CHUNK) * CHUNK
        total = total + pcw
        mine = mine + jnp.where(widv > wo, pcw, 0)
    base = plsc.cumsum(total) - total
    return base, total, mine


@functools.partial(
    pl.kernel,
    out_type=[jax.ShapeDtypeStruct((EP,), jnp.int32),
              jax.ShapeDtypeStruct((EP,), jnp.int32),
              jax.ShapeDtypeStruct((8, 128), jnp.int32)],
    mesh=_mesh,
    compiler_params=pltpu.CompilerParams(needs_layout_passes=False),
    scratch_types=[
        pltpu.VMEM((EPW,), jnp.int32),
        pltpu.VMEM((EPW,), jnp.int32),
        pltpu.VMEM((N,), jnp.int32),
        pltpu.VMEM((NW, 8, 128), jnp.int32),
        pltpu.VMEM((STAG,), jnp.int32),
        pltpu.VMEM((STAG,), jnp.int32),
        pltpu.VMEM((8, 128), jnp.int32),
        pltpu.SemaphoreType.DMA,
    ],
)
def _sc_pass2(src_hbm, dst_hbm, lvl_hbm, cnt_hbm, srcc_hbm, dstc_hbm,
              summ_hbm, srcblk, dstblk, lvl_v, cnts_v, sstag, dstag,
              sblk, sem):
    c = lax.axis_index("c")
    s = lax.axis_index("s")
    wid = s * NC + c
    woff = pl.multiple_of(wid * EPW, 8)
    pltpu.sync_copy(src_hbm.at[pl.ds(woff, EPW)], srcblk)
    pltpu.sync_copy(dst_hbm.at[pl.ds(woff, EPW)], dstblk)
    pltpu.sync_copy(lvl_hbm, lvl_v)
    pltpu.sync_copy(cnt_hbm, cnts_v)

    base, total, mine = _bucket_layout(cnts_v, wid)
    slot = base + mine

    for r in range(8):
        for q in range(8):
            sblk[r, pl.ds(q * 16, 16)] = jnp.zeros((16,), jnp.int32)
    sblk[0, pl.ds(0, 16)] = base
    sblk[1, pl.ds(0, 16)] = total

    @pl.when(wid == 0)
    def _pub():
        pltpu.sync_copy(sblk, summ_hbm)

    for l in range(1, 8):
        def _grp(g, ptr):
            s16 = srcblk[pl.ds(g * 16, 16)]
            d16 = dstblk[pl.ds(g * 16, 16)]
            dlev = plsc.load_gather(lvl_v, [d16])
            m = dlev == l
            mi = m.astype(jnp.int32)
            excl = plsc.cumsum(mi) - mi
            idx = excl + ptr
            plsc.store_scatter(sstag, [idx], s16, mask=m)
            plsc.store_scatter(dstag, [idx], d16, mask=m)
            return ptr + jnp.sum(mi)
        ptr = lax.fori_loop(0, NGRP, _grp, jnp.zeros((), jnp.int32))

        ones16 = jnp.zeros((16,), jnp.int32) + 1
        for g in range(CHUNK // 16):
            pidx = _iota16() + (ptr + g * 16)
            plsc.store_scatter(sstag, [pidx], jnp.zeros((16,), jnp.int32),
                               mask=ones16 == 1)
            plsc.store_scatter(dstag, [pidx],
                               jnp.zeros((16,), jnp.int32) + TRASH,
                               mask=ones16 == 1)

        myslot = jnp.sum(jnp.where(_iota16() == l, slot, 0))
        nch = (ptr + CHUNK - 1) // CHUNK

        def _out(j, _):
            o = pl.multiple_of(j * CHUNK, 8)
            go = pl.multiple_of(myslot + o, 8)
            pltpu.sync_copy(sstag.at[pl.ds(o, CHUNK)],
                            srcc_hbm.at[pl.ds(go, CHUNK)])
            pltpu.sync_copy(dstag.at[pl.ds(o, CHUNK)],
                            dstc_hbm.at[pl.ds(go, CHUNK)])
            return 0
        lax.fori_loop(0, nch, _out, 0)


@functools.partial(
    pl.kernel,
    out_type=jax.ShapeDtypeStruct((NC, NP, HID), jnp.float32),
    mesh=_mesh,
    compiler_params=pltpu.CompilerParams(needs_layout_passes=False),
    scratch_types=[
        pltpu.VMEM((16,), jnp.int32),
        pltpu.VMEM((8, 128), jnp.int32),
        pltpu.VMEM((CHUNK,), jnp.int32),
        pltpu.VMEM((CHUNK,), jnp.int32),
        pltpu.VMEM((CHUNK, HID), jnp.float32),
        pltpu.VMEM((ZR, HID), jnp.float32),
        pltpu.VMEM_SHARED((NP, HID), jnp.float32),
        pltpu.SemaphoreType.DMA,
    ],
)
def _sc_agg(hext_hbm, srcc_hbm, dstc_hbm, summ_hbm, oh_hbm, out_hbm,
            oh_v, summ_v, src_v, dst_v, rows_v, zero_v, acc_sh, sem):
    c = lax.axis_index("c")
    s = lax.axis_index("s")
    wid = s * NC + c
    pltpu.sync_copy(summ_hbm, summ_v)
    pltpu.sync_copy(oh_hbm, oh_v)
    oh = oh_v[...]

    def _zrow(i, _):
        for j in range(HID // 16):
            zero_v[i, pl.ds(j * 16, 16)] = jnp.zeros((16,), jnp.float32)
        return 0
    lax.fori_loop(0, ZR, _zrow, 0)

    def _zcpy(i, _):
        pltpu.sync_copy(zero_v, acc_sh.at[pl.ds(s * RPS + i * ZR, ZR)])
        return 0
    lax.fori_loop(0, RPS // ZR, _zcpy, 0)
    plsc.subcore_barrier()

    base16 = summ_v[0, pl.ds(0, 16)]
    total16 = summ_v[1, pl.ds(0, 16)]
    base_l = jnp.sum(jnp.where(oh == 1, base16, 0))
    nch_l = jnp.sum(jnp.where(oh == 1, total16, 0)) // CHUNK
    trips = (nch_l - wid + (NW - 1)) // NW

    def _chunk(j, _):
        off = pl.multiple_of(base_l + (wid + j * NW) * CHUNK, 8)
        pltpu.sync_copy(srcc_hbm.at[pl.ds(off, CHUNK)], src_v)
        pltpu.sync_copy(dstc_hbm.at[pl.ds(off, CHUNK)], dst_v)
        pltpu.async_copy(hext_hbm.at[src_v], rows_v, sem).wait()
        pltpu.sync_copy(rows_v, acc_sh.at[dst_v], add=True)
        return 0
    lax.fori_loop(0, trips, _chunk, 0)
    plsc.subcore_barrier()

    pltpu.sync_copy(acc_sh.at[pl.ds(s * RPS, RPS)],
                    out_hbm.at[c, pl.ds(s * RPS, RPS)])


NKCH = EPW // CHUNK


@functools.partial(
    pl.kernel,
    out_type=jax.ShapeDtypeStruct((NC, NP, HID), jnp.float32),
    mesh=_mesh,
    compiler_params=pltpu.CompilerParams(needs_layout_passes=False),
    scratch_types=[
        pltpu.VMEM((NKCH, CHUNK), jnp.int32),
        pltpu.VMEM((CHUNK, HID), jnp.float32),
        pltpu.VMEM((ZR, HID), jnp.float32),
        pltpu.VMEM_SHARED((NP, HID), jnp.float32),
        pltpu.SemaphoreType.DMA,
    ],
)
def _sc_deg(dst2_hbm, out_hbm, dstblk, ones_v, zero_v, acc_sh, sem):
    c = lax.axis_index("c")
    s = lax.axis_index("s")
    wid = s * NC + c
    pltpu.sync_copy(dst2_hbm.at[wid], dstblk)

    def _frow(i, _):
        for j in range(HID // 16):
            zero_v[i, pl.ds(j * 16, 16)] = jnp.zeros((16,), jnp.float32)
        return 0
    lax.fori_loop(0, ZR, _frow, 0)

    def _orow(i, _):
        for j in range(HID // 16):
            ones_v[i, pl.ds(j * 16, 16)] = jnp.ones((16,), jnp.float32)
        return 0
    lax.fori_loop(0, CHUNK, _orow, 0)

    def _zcpy(i, _):
        pltpu.sync_copy(zero_v, acc_sh.at[pl.ds(s * RPS + i * ZR, ZR)])
        return 0
    lax.fori_loop(0, RPS // ZR, _zcpy, 0)
    plsc.subcore_barrier()

    def _chunk(k, _):
        pltpu.sync_copy(ones_v, acc_sh.at[dstblk.at[k]], add=True)
        return 0
    lax.fori_loop(0, NKCH, _chunk, 0)
    plsc.subcore_barrier()

    pltpu.sync_copy(acc_sh.at[pl.ds(s * RPS, RPS)],
                    out_hbm.at[c, pl.ds(s * RPS, RPS)])


_RB = 1024
_NBLK = NP // _RB


def _leaky(x):
    return jnp.where(x >= 0, x, 0.1 * x)


def _tc_init_body(feat, delay, lvl, Wpi1, bpi1, Wpi2, bpi2, Ws1, bs1, Ws2,
                  bs2, hext_out, hself_out):
    hs = jnp.dot(feat[...], Ws1[...], preferred_element_type=jnp.float32)
    hs = _leaky(hs + bs1[...])
    hs = jnp.dot(hs, Ws2[...], preferred_element_type=jnp.float32) + bs2[...]
    hself_out[...] = hs

    hp = delay[...] * Wpi1[...]
    hp = _leaky(hp + bpi1[...])
    hp = jnp.dot(hp, Wpi2[...], preferred_element_type=jnp.float32) + bpi2[...]
    hext_out[...] = jnp.where(lvl[...] == 0, hp, 0.0)


def _tc_init(feat, delay, lvl2d, Wpi1, bpi1, Wpi2, bpi2, Ws1, bs1, Ws2, bs2):
    full = lambda shape: pl.BlockSpec(shape, lambda i: (0, 0))
    row = lambda w: pl.BlockSpec((_RB, w), lambda i: (i, 0))
    return pl.pallas_call(
        _tc_init_body,
        grid=(_NBLK,),
        in_specs=[row(HID), row(1), row(1),
                  full((1, 64)), full((1, 64)), full((64, HID)), full((1, HID)),
                  full((HID, 64)), full((1, 64)), full((64, HID)), full((1, HID))],
        out_specs=[row(HID), row(HID)],
        out_shape=[jax.ShapeDtypeStruct((NP, HID), jnp.float32),
                   jax.ShapeDtypeStruct((NP, HID), jnp.float32)],
    )(feat, delay, lvl2d, Wpi1, bpi1, Wpi2, bpi2, Ws1, bs1, Ws2, bs2)


def _tc_level_body(lref, accA, accB, degA, degB, hself, hext_in, lvl,
                   ispo, Wn1, bn1, Wn2, bn2, hext_out):
    lv = lref[0, 0]
    deg = jnp.maximum(degA[...] + degB[...], 1.0)
    neigh = (accA[...] + accB[...]) / deg
    hid = jnp.dot(neigh, Wn1[...], preferred_element_type=jnp.float32)
    hid = _leaky(hid + bn1[...])
    out = jnp.dot(hid, Wn2[...], preferred_element_type=jnp.float32) + bn2[...]
    out = out + hself[...]
    out = jnp.where(ispo[...] != 1, jnp.maximum(out, 0.0), out)
    hext_out[...] = jnp.where(lvl[...] == lv, out, hext_in[...])


def _tc_level(lval, accA, accB, degA, degB, hself, hext, lvl2d, ispo,
              Wn1, bn1, Wn2, bn2):
    full = lambda shape: pl.BlockSpec(shape, lambda i: (0, 0))
    row = lambda w: pl.BlockSpec((_RB, w), lambda i: (i, 0))
    return pl.pallas_call(
        _tc_level_body,
        grid=(_NBLK,),
        in_specs=[pl.BlockSpec(memory_space=pltpu.SMEM),
                  row(HID), row(HID), row(1), row(1), row(HID), row(HID),
                  row(1), row(1),
                  full((HID, 64)), full((1, 64)), full((64, HID)), full((1, HID))],
        out_specs=row(HID),
        out_shape=jax.ShapeDtypeStruct((NP, HID), jnp.float32),
    )(lval, accA, accB, degA, degB, hself, hext, lvl2d, ispo,
      Wn1, bn1, Wn2, bn2)


def kernel(feat, delay, is_po, edge_index, level_ids, Wpi1, bpi1, Wpi2, bpi2,
           Ws1, bs1, Ws2, bs2, Wn1, bn1, Wn2, bn2):
    src = edge_index[0]
    dst = edge_index[1]
    pad = NP - N
    featp = jnp.pad(feat, ((0, pad), (0, 0)))
    delayp = jnp.pad(delay, ((0, pad), (0, 0)))
    ispop = jnp.pad(is_po, ((0, pad), (0, 0)))
    lvlp = jnp.pad(level_ids, (0, pad), constant_values=99)[:, None]

    hext, h_self = _tc_init(featp, delayp, lvlp,
                            Wpi1, bpi1[None, :], Wpi2, bpi2[None, :],
                            Ws1, bs1[None, :], Ws2, bs2[None, :])

    cnts = _sc_pass1(dst, level_ids)
    srcc, dstc, summ = _sc_pass2(src, dst, level_ids, cnts)
    degp = _sc_deg(dst.reshape(NW, NKCH, CHUNK))
    degA = degp[0, :, 0:1]
    degB = degp[1, :, 0:1]

    bn1r = bn1[None, :]
    bn2r = bn2[None, :]
    for l in range(1, NLVL):
        oh = (jnp.arange(16, dtype=jnp.int32) == l).astype(jnp.int32)
        acc = _sc_agg(hext, srcc, dstc, summ, oh)
        lval = jnp.full((1, 1), l, dtype=jnp.int32)
        hext = _tc_level(lval, acc[0], acc[1], degA, degB, h_self, hext,
                         lvlp, ispop, Wn1, bn1r, Wn2, bn2r)
    return hext[:N, :HID]
